# baseline (speedup 1.0000x reference)
import functools

import numpy as np
import jax
import jax.numpy as jnp
from jax import lax
from jax.experimental import pallas as pl
from jax.experimental.pallas import tpu as pltpu
from jax.experimental.pallas import tpu_sc as plsc

N_ROWS = 16384
N_COLS = 1000
BR = 512
G = N_ROWS // BR
N_TILES = 16
CHUNK = N_ROWS // N_TILES
L = 16
NV = CHUNK // L


def _tc_body(probs_ref, labels_ref, conf_ref, acc_ref):
    x = probs_ref[...]
    conf = jnp.max(x, axis=1)
    idx = lax.broadcasted_iota(jnp.int32, x.shape, 1)
    cand = jnp.where(x == conf[:, None], idx, N_COLS)
    pred = jnp.min(cand, axis=1)
    lab = labels_ref[0, 0, :]
    acc = (pred == lab).astype(jnp.float32)
    conf_ref[0, 0, :] = conf
    acc_ref[0, 0, :] = acc


def _tc_stage(probs, labels3):
    conf3, acc3 = pl.pallas_call(
        _tc_body,
        grid=(G,),
        in_specs=[
            pl.BlockSpec((BR, N_COLS), lambda i: (i, 0)),
            pl.BlockSpec((1, 1, BR), lambda i: (i, 0, 0)),
        ],
        out_specs=[
            pl.BlockSpec((1, 1, BR), lambda i: (i, 0, 0)),
            pl.BlockSpec((1, 1, BR), lambda i: (i, 0, 0)),
        ],
        out_shape=[
            jax.ShapeDtypeStruct((G, 1, BR), jnp.float32),
            jax.ShapeDtypeStruct((G, 1, BR), jnp.float32),
        ],
    )(probs, labels3)
    return conf3.reshape(-1), acc3.reshape(-1)


def _sc_stage(conf, acc):
    mesh = plsc.VectorSubcoreMesh(core_axis_name="c", subcore_axis_name="s")

    @functools.partial(
        pl.kernel,
        mesh=mesh,
        out_type=jax.ShapeDtypeStruct((8 * N_TILES, 16), jnp.float32),
        scratch_types=[
            pltpu.VMEM((CHUNK,), jnp.float32),
            pltpu.VMEM((CHUNK,), jnp.float32),
            pltpu.VMEM((8, 16), jnp.float32),
        ],
    )
    def k(conf_hbm, acc_hbm, out_hbm, conf_v, acc_v, hist):
        cid = lax.axis_index("c")
        sid = lax.axis_index("s")
        zeros = jnp.zeros((L,), jnp.float32)
        lanes = lax.broadcasted_iota(jnp.int32, (L,), 0)
        step = jnp.float32(np.float32(1.0 / 15.0))
        lf = lanes.astype(jnp.float32)
        blow = jnp.where(lanes == 15, 2.0, lf * step)
        bhigh = jnp.where(lanes == 15, 2.0,
                          jnp.where(lanes == 14, 1.0, (lf + 1.0) * step))

        @pl.when(cid == 0)
        def _compute():
            base = sid * CHUNK
            pltpu.sync_copy(conf_hbm.at[pl.ds(base, CHUNK)], conf_v)
            pltpu.sync_copy(acc_hbm.at[pl.ds(base, CHUNK)], acc_v)

            def body(i, carry):
                cnt_s, acc_s, cf_s = carry
                off = pl.multiple_of(i * L, L)
                cvec = conf_v[pl.ds(off, L)]
                avec = acc_v[pl.ds(off, L)]
                for e in range(L):
                    c_e = cvec[e]
                    a_e = avec[e]
                    m = (c_e > blow) & (c_e <= bhigh)
                    inc = jnp.where(m, 1.0, 0.0)
                    cnt_s = cnt_s + inc
                    acc_s = acc_s + inc * a_e
                    cf_s = cf_s + inc * c_e
                return cnt_s, acc_s, cf_s

            cnt_s, acc_s, cf_s = lax.fori_loop(0, NV, body,
                                               (zeros, zeros, zeros))
            hist[0, pl.ds(0, 16)] = cnt_s
            hist[1, pl.ds(0, 16)] = acc_s
            hist[2, pl.ds(0, 16)] = cf_s
            for r in range(3, 8):
                hist[r, pl.ds(0, 16)] = zeros
            pltpu.sync_copy(hist, out_hbm.at[pl.ds(sid * 8, 8)])

    return k(conf, acc)


def _combine_body(part_ref, out_ref):
    x = part_ref[...]
    rows = lax.broadcasted_iota(jnp.int32, x.shape, 0)
    q = rows % 8
    cnt = jnp.sum(jnp.where(q == 0, x, 0.0), axis=0)
    accs = jnp.sum(jnp.where(q == 1, x, 0.0), axis=0)
    cfs = jnp.sum(jnp.where(q == 2, x, 0.0), axis=0)
    n = jnp.float32(N_ROWS)
    prop = cnt / n
    safe = jnp.maximum(cnt, 1.0)
    gap = jnp.abs(cfs / safe - accs / safe) * prop
    gap = jnp.where(cnt > 0, gap, 0.0)
    out_ref[0] = jnp.sum(gap)


def _combine_stage(partials):
    out = pl.pallas_call(
        _combine_body,
        out_specs=pl.BlockSpec(memory_space=pltpu.SMEM),
        out_shape=jax.ShapeDtypeStruct((1,), jnp.float32),
    )(partials)
    return out


def kernel(probs, labels):
    labels3 = labels.astype(jnp.int32).reshape(G, 1, BR)
    conf, acc = _tc_stage(probs, labels3)
    partials = _sc_stage(conf, acc)
    return _combine_stage(partials)
